# Initial kernel scaffold; baseline (speedup 1.0000x reference)
import functools

import jax

jax.config.update("jax_enable_x64", True)

import jax.numpy as jnp
from jax import lax
from jax.experimental import pallas as pl
from jax.experimental.pallas import tpu as pltpu
from jax.experimental.pallas import tpu_sc as plsc

NC, NS, LANES = 2, 16, 16
NW = NC * NS

VOCAB_W, VOCAB_C, VOCAB_T = 100001, 129, 46
PAD_W, PAD_C, PAD_T = 100016, 144, 64

CHUNK = 6400


def _sc_gather_call(s_words, c_words, t_words, wt_pad, ct_pad, tt_pad):
    sw_n, cw_n, tw_n = s_words.shape[0], c_words.shape[0], t_words.shape[0]
    mesh = plsc.VectorSubcoreMesh(core_axis_name="c", subcore_axis_name="s")

    @functools.partial(
        pl.kernel,
        out_type=(
            jax.ShapeDtypeStruct((sw_n,), jnp.int32),
            jax.ShapeDtypeStruct((cw_n,), jnp.int32),
            jax.ShapeDtypeStruct((tw_n,), jnp.int32),
        ),
        mesh=mesh,
        scratch_types=[
            pltpu.VMEM((PAD_W,), jnp.int32),
            pltpu.VMEM((PAD_C,), jnp.int32),
            pltpu.VMEM((PAD_T,), jnp.int32),
            pltpu.VMEM((CHUNK,), jnp.int32),
            pltpu.VMEM((CHUNK,), jnp.int32),
        ],
    )
    def run(s_hbm, c_hbm, t_hbm, wt_hbm, ct_hbm, tt_hbm,
            so_hbm, co_hbm, to_hbm,
            wt_v, ct_v, tt_v, in_v, out_v):
        wid = lax.axis_index("s") * NC + lax.axis_index("c")

        pltpu.sync_copy(wt_hbm, wt_v)
        pltpu.sync_copy(ct_hbm, ct_v)
        pltpu.sync_copy(tt_hbm, tt_v)

        parity = lax.iota(jnp.int32, LANES) & 1

        def phase(in_hbm, out_hbm, table_v, zero_slot, total_words):
            per_tile = total_words // NW
            nchunks = per_tile // CHUNK
            base = wid * per_tile
            addv = parity * zero_slot

            def chunk_body(ci, _):
                off = base + ci * CHUNK
                pltpu.sync_copy(in_hbm.at[pl.ds(off, CHUNK)], in_v)

                def vec_body(i, _):
                    v = in_v[pl.ds(i * LANES, LANES)]
                    out_v[pl.ds(i * LANES, LANES)] = plsc.load_gather(
                        table_v, [v + addv])
                    return 0

                lax.fori_loop(0, CHUNK // LANES, vec_body, 0)
                pltpu.sync_copy(out_v, out_hbm.at[pl.ds(off, CHUNK)])
                return 0

            lax.fori_loop(0, nchunks, chunk_body, 0)

        phase(s_hbm, so_hbm, wt_v, VOCAB_W, sw_n)
        phase(c_hbm, co_hbm, ct_v, VOCAB_C, cw_n)
        phase(t_hbm, to_hbm, tt_v, VOCAB_T, tw_n)

    return run(s_words, c_words, t_words, wt_pad, ct_pad, tt_pad)


def kernel(sentence_tensor, char_tensor, tag_string_tensor,
           word_table, char_table, tag_table):
    s_words = lax.bitcast_convert_type(sentence_tensor, jnp.int32).reshape(-1)
    c_words = lax.bitcast_convert_type(char_tensor, jnp.int32).reshape(-1)
    t_words = lax.bitcast_convert_type(tag_string_tensor, jnp.int32).reshape(-1)

    def pad_table(tb, pad_len):
        out = jnp.zeros((pad_len,), jnp.int32)
        return out.at[: tb.shape[0]].set(tb.astype(jnp.int32))

    wt_pad = pad_table(word_table, PAD_W)
    ct_pad = pad_table(char_table, PAD_C)
    tt_pad = pad_table(tag_table, PAD_T)

    so, co, to = _sc_gather_call(s_words, c_words, t_words, wt_pad, ct_pad, tt_pad)

    def to64(words, shape):
        return lax.bitcast_convert_type(
            words.reshape(shape + (2,)), jnp.int64)

    return (
        to64(so, sentence_tensor.shape),
        to64(co, char_tensor.shape),
        to64(to, tag_string_tensor.shape),
    )


# speedup vs baseline: 12.5503x; 12.5503x over previous
import functools

import jax

jax.config.update("jax_enable_x64", True)

import jax.numpy as jnp
from jax import lax
from jax.experimental import pallas as pl
from jax.experimental.pallas import tpu as pltpu
from jax.experimental.pallas import tpu_sc as plsc

NC, NS, LANES = 2, 16, 16
NW = NC * NS

VOCAB_W, VOCAB_C, VOCAB_T = 100001, 129, 46
PAD_W, PAD_C, PAD_T = 100016, 144, 64

CHUNK = 6400


def _sc_gather_call(s_words, c_words, t_words, wt_pad, ct_pad, tt_pad):
    sw_n, cw_n, tw_n = s_words.shape[0], c_words.shape[0], t_words.shape[0]
    mesh = plsc.VectorSubcoreMesh(core_axis_name="c", subcore_axis_name="s")

    @functools.partial(
        pl.kernel,
        out_type=(
            jax.ShapeDtypeStruct((sw_n,), jnp.int32),
            jax.ShapeDtypeStruct((cw_n,), jnp.int32),
            jax.ShapeDtypeStruct((tw_n,), jnp.int32),
        ),
        mesh=mesh,
        scratch_types=[
            pltpu.VMEM((PAD_W,), jnp.int32),
            pltpu.VMEM((PAD_C,), jnp.int32),
            pltpu.VMEM((PAD_T,), jnp.int32),
            pltpu.VMEM((CHUNK,), jnp.int32),
            pltpu.VMEM((CHUNK,), jnp.int32),
        ],
        compiler_params=pltpu.CompilerParams(needs_layout_passes=False),
    )
    def run(s_hbm, c_hbm, t_hbm, wt_hbm, ct_hbm, tt_hbm,
            so_hbm, co_hbm, to_hbm,
            wt_v, ct_v, tt_v, in_v, out_v):
        wid = lax.axis_index("s") * NC + lax.axis_index("c")

        pltpu.sync_copy(wt_hbm, wt_v)
        pltpu.sync_copy(ct_hbm, ct_v)
        pltpu.sync_copy(tt_hbm, tt_v)

        parity = lax.iota(jnp.int32, LANES) & 1

        def phase(in_hbm, out_hbm, table_v, zero_slot, total_words):
            per_tile = total_words // NW
            nchunks = per_tile // CHUNK
            base = wid * jnp.int32(per_tile)
            addv = parity * jnp.int32(zero_slot)

            def chunk_body(ci, _):
                off = base + ci * jnp.int32(CHUNK)
                pltpu.sync_copy(in_hbm.at[pl.ds(off, CHUNK)], in_v)

                def vec_body(i, _):
                    v = in_v[pl.ds(i * jnp.int32(LANES), LANES)]
                    out_v[pl.ds(i * jnp.int32(LANES), LANES)] = plsc.load_gather(
                        table_v, [v + addv])
                    return jnp.int32(0)

                lax.fori_loop(jnp.int32(0), jnp.int32(CHUNK // LANES),
                              vec_body, jnp.int32(0))
                pltpu.sync_copy(out_v, out_hbm.at[pl.ds(off, CHUNK)])
                return jnp.int32(0)

            lax.fori_loop(jnp.int32(0), jnp.int32(nchunks), chunk_body,
                          jnp.int32(0))

        phase(s_hbm, so_hbm, wt_v, VOCAB_W, sw_n)
        phase(c_hbm, co_hbm, ct_v, VOCAB_C, cw_n)
        phase(t_hbm, to_hbm, tt_v, VOCAB_T, tw_n)

    return run(s_words, c_words, t_words, wt_pad, ct_pad, tt_pad)


def kernel(sentence_tensor, char_tensor, tag_string_tensor,
           word_table, char_table, tag_table):
    s_words = lax.bitcast_convert_type(sentence_tensor, jnp.int32).reshape(-1)
    c_words = lax.bitcast_convert_type(char_tensor, jnp.int32).reshape(-1)
    t_words = lax.bitcast_convert_type(tag_string_tensor, jnp.int32).reshape(-1)

    def pad_table(tb, pad_len):
        out = jnp.zeros((pad_len,), jnp.int32)
        return out.at[: tb.shape[0]].set(tb.astype(jnp.int32))

    wt_pad = pad_table(word_table, PAD_W)
    ct_pad = pad_table(char_table, PAD_C)
    tt_pad = pad_table(tag_table, PAD_T)

    so, co, to = _sc_gather_call(s_words, c_words, t_words, wt_pad, ct_pad, tt_pad)

    def to64(words, shape):
        return lax.bitcast_convert_type(
            words.reshape(shape + (2,)), jnp.int64)

    return (
        to64(so, sentence_tensor.shape),
        to64(co, char_tensor.shape),
        to64(to, tag_string_tensor.shape),
    )
